# Initial kernel scaffold; baseline (speedup 1.0000x reference)
import jax
import jax.numpy as jnp
from jax.experimental import pallas as pl


def kernel(x, edge_index, W1, b1, W2, b2, W3, b3):
    raise NotImplementedError("write your pallas kernel here")


# speedup vs baseline: 3.5672x; 3.5672x over previous
import functools

import jax
import jax.numpy as jnp
from jax import lax
from jax.experimental import pallas as pl
from jax.experimental.pallas import tpu as pltpu
from jax.experimental.pallas import tpu_sc as plsc

_N = 10000
_E = 320000
_D = 128

_NC = 2
_NS = 16
_NW = _NC * _NS

_N_PAD = 10240
_ROWS_PER_TILE = _N_PAD // _NS
_CH = 128
_NCH = 79
_EPT = _CH * _NCH
_E_PAD = _EPT * _NW

_mesh = plsc.VectorSubcoreMesh(
    core_axis_name="c", subcore_axis_name="s", num_cores=_NC, num_subcores=_NS
)
_sc_params = pltpu.CompilerParams(needs_layout_passes=False)


def _deg_body(src_hbm, dst_hbm, odeg_hbm, ideg_hbm, src_v, dst_v, od_v, id_v):
    c = lax.axis_index("c")
    s = lax.axis_index("s")
    wid = c * _NS + s
    base = wid * _EPT
    pltpu.sync_copy(src_hbm.at[pl.ds(base, _EPT)], src_v)
    pltpu.sync_copy(dst_hbm.at[pl.ds(base, _EPT)], dst_v)

    zeros16 = jnp.zeros((16,), jnp.float32)

    def zero_body(i, carry):
        od_v[pl.ds(i * 16, 16)] = zeros16
        id_v[pl.ds(i * 16, 16)] = zeros16
        return carry

    lax.fori_loop(0, _N_PAD // 16, zero_body, 0)

    ones16 = jnp.ones((16,), jnp.float32)

    def hist_body(i, carry):
        si = src_v[pl.ds(i * 16, 16)]
        di = dst_v[pl.ds(i * 16, 16)]
        plsc.addupdate_scatter(od_v, [si], ones16)
        plsc.addupdate_scatter(id_v, [di], ones16)
        return carry

    lax.fori_loop(0, _EPT // 16, hist_body, 0)

    pltpu.sync_copy(od_v, odeg_hbm.at[wid])
    pltpu.sync_copy(id_v, ideg_hbm.at[wid])


_deg_kernel = pl.kernel(
    _deg_body,
    out_type=(
        jax.ShapeDtypeStruct((_NW, _N_PAD), jnp.float32),
        jax.ShapeDtypeStruct((_NW, _N_PAD), jnp.float32),
    ),
    mesh=_mesh,
    compiler_params=_sc_params,
    scratch_types=[
        pltpu.VMEM((_EPT,), jnp.int32),
        pltpu.VMEM((_EPT,), jnp.int32),
        pltpu.VMEM((_N_PAD,), jnp.float32),
        pltpu.VMEM((_N_PAD,), jnp.float32),
    ],
)


def _agg_body(m_hbm, src_hbm, dst_hbm, out_hbm, sidx, didx, rows, zblk, agg_sh):
    c = lax.axis_index("c")
    s = lax.axis_index("s")
    wid = c * _NS + s
    ebase = wid * _EPT

    zeros16 = jnp.zeros((16,), jnp.float32)

    def zero_body(i, carry):
        zblk[i // 8, pl.ds((i % 8) * 16, 16)] = zeros16
        return carry

    lax.fori_loop(0, 128 * 8, zero_body, 0)
    for k in range(_ROWS_PER_TILE // 128):
        pltpu.sync_copy(zblk, agg_sh.at[pl.ds(s * _ROWS_PER_TILE + k * 128, 128)])
    plsc.subcore_barrier()

    def chunk_body(j, carry):
        pltpu.sync_copy(src_hbm.at[pl.ds(ebase + j * _CH, _CH)], sidx.at[0])
        pltpu.sync_copy(dst_hbm.at[pl.ds(ebase + j * _CH, _CH)], didx.at[0])
        pltpu.sync_copy(m_hbm.at[sidx.at[0]], rows)
        pltpu.sync_copy(rows, agg_sh.at[didx.at[0]], add=True)
        return carry

    lax.fori_loop(0, _NCH, chunk_body, 0)
    plsc.subcore_barrier()

    pltpu.sync_copy(
        agg_sh.at[pl.ds(s * _ROWS_PER_TILE, _ROWS_PER_TILE)],
        out_hbm.at[c, pl.ds(s * _ROWS_PER_TILE, _ROWS_PER_TILE)],
    )


_agg_kernel = pl.kernel(
    _agg_body,
    out_type=jax.ShapeDtypeStruct((_NC, _N_PAD, _D), jnp.float32),
    mesh=_mesh,
    compiler_params=_sc_params,
    scratch_types=[
        pltpu.VMEM((1, _CH), jnp.int32),
        pltpu.VMEM((1, _CH), jnp.int32),
        pltpu.VMEM((_CH, _D), jnp.float32),
        pltpu.VMEM((128, _D), jnp.float32),
        pltpu.VMEM_SHARED((_N_PAD, _D), jnp.float32),
    ],
)


_BLK = 512
_GRID = _N_PAD // _BLK


def _norm_body(od_ref, id_ref, x_ref, m1_ref, ns_ref, nd_ref):
    od = jnp.sum(od_ref[...], axis=0)
    idg = jnp.sum(id_ref[...], axis=0)
    ns = lax.rsqrt(jnp.where(od > 0, od, 1.0))
    nd = lax.rsqrt(jnp.where(idg > 0, idg, 1.0))
    ns_ref[...] = ns[:, None]
    nd_ref[...] = nd[:, None]
    m1_ref[...] = x_ref[...] * ns[:, None]


_norm_kernel = pl.pallas_call(
    _norm_body,
    grid=(_GRID,),
    in_specs=[
        pl.BlockSpec((_NW, _BLK), lambda i: (0, i)),
        pl.BlockSpec((_NW, _BLK), lambda i: (0, i)),
        pl.BlockSpec((_BLK, _D), lambda i: (i, 0)),
    ],
    out_specs=[
        pl.BlockSpec((_BLK, _D), lambda i: (i, 0)),
        pl.BlockSpec((_BLK, 1), lambda i: (i, 0)),
        pl.BlockSpec((_BLK, 1), lambda i: (i, 0)),
    ],
    out_shape=[
        jax.ShapeDtypeStruct((_N_PAD, _D), jnp.float32),
        jax.ShapeDtypeStruct((_N_PAD, 1), jnp.float32),
        jax.ShapeDtypeStruct((_N_PAD, 1), jnp.float32),
    ],
)


def _make_layer_kernel(relu, scale_next):
    def body(p0_ref, p1_ref, nd_ref, ns_ref, w_ref, b_ref, out_ref):
        agg = (p0_ref[...] + p1_ref[...]) * nd_ref[...]
        h = jnp.dot(agg, w_ref[...], preferred_element_type=jnp.float32)
        h = h + b_ref[...]
        if relu:
            h = jnp.maximum(h, 0.0)
        if scale_next:
            h = h * ns_ref[...]
        out_ref[...] = h

    return pl.pallas_call(
        body,
        grid=(_GRID,),
        in_specs=[
            pl.BlockSpec((_BLK, _D), lambda i: (i, 0)),
            pl.BlockSpec((_BLK, _D), lambda i: (i, 0)),
            pl.BlockSpec((_BLK, 1), lambda i: (i, 0)),
            pl.BlockSpec((_BLK, 1), lambda i: (i, 0)),
            pl.BlockSpec((_D, _D), lambda i: (0, 0)),
            pl.BlockSpec((1, _D), lambda i: (0, 0)),
        ],
        out_specs=pl.BlockSpec((_BLK, _D), lambda i: (i, 0)),
        out_shape=jax.ShapeDtypeStruct((_N_PAD, _D), jnp.float32),
    )


_layer_mid = _make_layer_kernel(relu=True, scale_next=True)
_layer_last = _make_layer_kernel(relu=False, scale_next=False)


@jax.jit
def kernel(x, edge_index, W1, b1, W2, b2, W3, b3):
    src = edge_index[0].astype(jnp.int32)
    dst = edge_index[1].astype(jnp.int32)
    pad_idx = jnp.full((_E_PAD - _E,), _N_PAD - 1, jnp.int32)
    src_p = jnp.concatenate([src, pad_idx])
    dst_p = jnp.concatenate([dst, pad_idx])

    x_pad = jnp.zeros((_N_PAD, _D), jnp.float32).at[:_N].set(x)

    odeg, ideg = _deg_kernel(src_p, dst_p)
    m1, ns, nd = _norm_kernel(odeg, ideg, x_pad)

    p = _agg_kernel(m1, src_p, dst_p)
    m2 = _layer_mid(p[0], p[1], nd, ns, W1, b1.reshape(1, _D))
    p = _agg_kernel(m2, src_p, dst_p)
    m3 = _layer_mid(p[0], p[1], nd, ns, W2, b2.reshape(1, _D))
    p = _agg_kernel(m3, src_p, dst_p)
    out = _layer_last(p[0], p[1], nd, ns, W3, b3.reshape(1, _D))
    return out[:_N]
